# Initial kernel scaffold; baseline (speedup 1.0000x reference)
import jax
import jax.numpy as jnp
from jax.experimental import pallas as pl


def kernel(X, edge_index, W, b):
    raise NotImplementedError("write your pallas kernel here")


# speedup vs baseline: 28.7569x; 28.7569x over previous
import functools

import jax
import jax.numpy as jnp
from jax import lax
from jax.experimental import pallas as pl
from jax.experimental.pallas import tpu as pltpu
from jax.experimental.pallas import tpu_sc as plsc

N = 10000
E = 320000
D = 128

NC = 2
NS = 16
NW = NC * NS
L = 16

N_PAD = 10240
STRIPE = N_PAD // NS
E_PAD = NW * 10240
EPT = E_PAD // NW
KJ = EPT // 128


def _matvec_body(x_ref, w_ref, o_ref):
    o_ref[...] = lax.dot_general(
        x_ref[...], w_ref[...],
        dimension_numbers=(((1,), (1,)), ((), ())),
        preferred_element_type=jnp.float32,
    )


def _tc_matvec(X, W):
    return pl.pallas_call(
        _matvec_body,
        out_shape=jax.ShapeDtypeStruct((N, 1), jnp.float32),
    )(X, W)


def _sc_edge_body(y_hbm, src_hbm, dst_hbm, s_out, deg_out,
                  y_v, src_v, dst_v, vals_v, ones_v, tmp_v, s_sh, deg_sh):
    cid = lax.axis_index("c")
    sid = lax.axis_index("s")
    wid = cid * NS + sid

    pltpu.sync_copy(y_hbm, y_v)
    pltpu.sync_copy(src_hbm.at[wid], src_v)
    pltpu.sync_copy(dst_hbm.at[wid], dst_v)

    zero = jnp.zeros((L,), jnp.float32)
    for i in range(STRIPE // L):
        tmp_v[pl.ds(i * L, L)] = zero
    pltpu.sync_copy(tmp_v, s_sh.at[pl.ds(sid * STRIPE, STRIPE)])
    pltpu.sync_copy(tmp_v, deg_sh.at[pl.ds(sid * STRIPE, STRIPE)])

    one = jnp.ones((L,), jnp.float32)
    for i in range(128 // L):
        ones_v[pl.ds(i * L, L)] = one

    def gather_chunk(j, carry):
        for i in range(128 // L):
            idx = src_v[j, pl.ds(i * L, L)]
            vals_v[j, pl.ds(i * L, L)] = plsc.load_gather(y_v, [idx])
        return carry

    lax.fori_loop(0, KJ, gather_chunk, 0)

    plsc.subcore_barrier()

    def scatter_chunk(j, carry):
        pltpu.sync_copy(vals_v.at[j], s_sh.at[dst_v.at[j]], add=True)
        pltpu.sync_copy(ones_v, deg_sh.at[dst_v.at[j]], add=True)
        return carry

    lax.fori_loop(0, KJ, scatter_chunk, 0)

    plsc.subcore_barrier()

    pltpu.sync_copy(s_sh.at[pl.ds(sid * STRIPE, STRIPE)], tmp_v)
    pltpu.sync_copy(tmp_v, s_out.at[cid, sid])
    pltpu.sync_copy(deg_sh.at[pl.ds(sid * STRIPE, STRIPE)], tmp_v)
    pltpu.sync_copy(tmp_v, deg_out.at[cid, sid])


_sc_edge = functools.partial(
    pl.kernel,
    out_type=(
        jax.ShapeDtypeStruct((NC, NS, STRIPE), jnp.float32),
        jax.ShapeDtypeStruct((NC, NS, STRIPE), jnp.float32),
    ),
    mesh=plsc.VectorSubcoreMesh(
        core_axis_name="c", subcore_axis_name="s",
        num_cores=NC, num_subcores=NS,
    ),
    compiler_params=pltpu.CompilerParams(
        needs_layout_passes=False,
        use_tc_tiling_on_sc=False,
    ),
    scratch_types=[
        pltpu.VMEM((N_PAD,), jnp.float32),
        pltpu.VMEM((KJ, 128), jnp.int32),
        pltpu.VMEM((KJ, 128), jnp.int32),
        pltpu.VMEM((KJ, 128), jnp.float32),
        pltpu.VMEM((128,), jnp.float32),
        pltpu.VMEM((STRIPE,), jnp.float32),
        pltpu.VMEM_SHARED((N_PAD,), jnp.float32),
        pltpu.VMEM_SHARED((N_PAD,), jnp.float32),
    ],
)(_sc_edge_body)


def _finale_body(s_ref, d_ref, b_ref, o_ref):
    s = s_ref[0] + s_ref[1]
    d = jnp.maximum(d_ref[0] + d_ref[1], 1.0)
    o_ref[...] = jax.nn.sigmoid(s / d + b_ref[0, 0])


def _tc_finale(s_part, deg_part, b):
    return pl.pallas_call(
        _finale_body,
        in_specs=[
            pl.BlockSpec(memory_space=pltpu.VMEM),
            pl.BlockSpec(memory_space=pltpu.VMEM),
            pl.BlockSpec(memory_space=pltpu.SMEM),
        ],
        out_specs=pl.BlockSpec(memory_space=pltpu.VMEM),
        out_shape=jax.ShapeDtypeStruct((8, N_PAD // 8), jnp.float32),
    )(s_part.reshape(NC, 8, N_PAD // 8), deg_part.reshape(NC, 8, N_PAD // 8),
      b.reshape(1, 1))


@jax.jit
def kernel(X, edge_index, W, b):
    y = _tc_matvec(X, W)[:, 0]
    y = jnp.concatenate([y, jnp.zeros((N_PAD - N,), jnp.float32)])

    pad = E_PAD - E
    src = jnp.concatenate([edge_index[0], jnp.arange(pad, dtype=jnp.int32) % N])
    dst = jnp.concatenate(
        [edge_index[1], N + (jnp.arange(pad, dtype=jnp.int32) % (N_PAD - N))])
    src = src.reshape(NW, KJ, 128)
    dst = dst.reshape(NW, KJ, 128)

    s_part, deg_part = _sc_edge(y, src, dst)

    out = _tc_finale(s_part, deg_part, b)
    return out.reshape(N_PAD)[:N]


# speedup vs baseline: 38.5854x; 1.3418x over previous
import functools

import jax
import jax.numpy as jnp
from jax import lax
from jax.experimental import pallas as pl
from jax.experimental.pallas import tpu as pltpu
from jax.experimental.pallas import tpu_sc as plsc

N = 10000
E = 320000
D = 128

NC = 2
NS = 16
NW = NC * NS
L = 16

EPT = E // NW
N_ACC = 10240
STRIPE = N_ACC // NS


def _matvec_body(x_ref, w_ref, o_ref):
    o_ref[...] = lax.dot_general(
        x_ref[...], w_ref[...],
        dimension_numbers=(((1,), (1,)), ((), ())),
        preferred_element_type=jnp.float32,
    )


def _tc_matvec(X, W):
    return pl.pallas_call(
        _matvec_body,
        out_shape=jax.ShapeDtypeStruct((N, 1), jnp.float32),
    )(X, W)


def _sc_edge_body(y_hbm, edges_hbm, s_out, deg_out,
                  y_v, src_v, dst_v, vals_v, ones_v, tmp_v, s_sh, deg_sh):
    cid = lax.axis_index("c")
    sid = lax.axis_index("s")
    wid = cid * NS + sid

    pltpu.sync_copy(y_hbm, y_v)
    pltpu.sync_copy(edges_hbm.at[0, pl.ds(wid * EPT, EPT)], src_v)
    pltpu.sync_copy(edges_hbm.at[1, pl.ds(wid * EPT, EPT)], dst_v)

    zero = jnp.zeros((L,), jnp.float32)
    for i in range(STRIPE // L):
        tmp_v[pl.ds(i * L, L)] = zero
    pltpu.sync_copy(tmp_v, s_sh.at[pl.ds(sid * STRIPE, STRIPE)])
    pltpu.sync_copy(tmp_v, deg_sh.at[pl.ds(sid * STRIPE, STRIPE)])

    one = jnp.ones((L,), jnp.float32)

    def gather_chunk(i, carry):
        idx = src_v[pl.ds(i * L, L)]
        vals_v[pl.ds(i * L, L)] = plsc.load_gather(y_v, [idx])
        ones_v[pl.ds(i * L, L)] = one
        return carry

    lax.fori_loop(0, EPT // L, gather_chunk, 0)

    plsc.subcore_barrier()

    pltpu.sync_copy(vals_v, s_sh.at[dst_v], add=True)
    pltpu.sync_copy(ones_v, deg_sh.at[dst_v], add=True)

    plsc.subcore_barrier()

    pltpu.sync_copy(s_sh.at[pl.ds(sid * STRIPE, STRIPE)], tmp_v)
    pltpu.sync_copy(tmp_v, s_out.at[cid, sid])
    pltpu.sync_copy(deg_sh.at[pl.ds(sid * STRIPE, STRIPE)], tmp_v)
    pltpu.sync_copy(tmp_v, deg_out.at[cid, sid])


_sc_edge = functools.partial(
    pl.kernel,
    out_type=(
        jax.ShapeDtypeStruct((NC, NS, STRIPE), jnp.float32),
        jax.ShapeDtypeStruct((NC, NS, STRIPE), jnp.float32),
    ),
    mesh=plsc.VectorSubcoreMesh(
        core_axis_name="c", subcore_axis_name="s",
        num_cores=NC, num_subcores=NS,
    ),
    compiler_params=pltpu.CompilerParams(
        needs_layout_passes=False,
        use_tc_tiling_on_sc=False,
    ),
    scratch_types=[
        pltpu.VMEM((N,), jnp.float32),
        pltpu.VMEM((EPT,), jnp.int32),
        pltpu.VMEM((EPT,), jnp.int32),
        pltpu.VMEM((EPT,), jnp.float32),
        pltpu.VMEM((EPT,), jnp.float32),
        pltpu.VMEM((STRIPE,), jnp.float32),
        pltpu.VMEM_SHARED((N_ACC,), jnp.float32),
        pltpu.VMEM_SHARED((N_ACC,), jnp.float32),
    ],
)(_sc_edge_body)


def _finale_body(s_ref, d_ref, b_ref, o_ref):
    s = s_ref[0] + s_ref[1]
    d = jnp.maximum(d_ref[0] + d_ref[1], 1.0)
    o_ref[...] = jax.nn.sigmoid(s / d + b_ref[0, 0])


def _tc_finale(s_part, deg_part, b):
    return pl.pallas_call(
        _finale_body,
        in_specs=[
            pl.BlockSpec(memory_space=pltpu.VMEM),
            pl.BlockSpec(memory_space=pltpu.VMEM),
            pl.BlockSpec(memory_space=pltpu.SMEM),
        ],
        out_specs=pl.BlockSpec(memory_space=pltpu.VMEM),
        out_shape=jax.ShapeDtypeStruct((8, N_ACC // 8), jnp.float32),
    )(s_part.reshape(NC, 8, N_ACC // 8), deg_part.reshape(NC, 8, N_ACC // 8),
      b.reshape(1, 1))


@jax.jit
def kernel(X, edge_index, W, b):
    y = _tc_matvec(X, W).reshape(N)

    s_part, deg_part = _sc_edge(y, edge_index)

    out = _tc_finale(s_part, deg_part, b)
    return out.reshape(N_ACC)[:N]
